# Initial kernel scaffold; baseline (speedup 1.0000x reference)
import functools

import jax
import jax.numpy as jnp
from jax import lax
from jax.experimental import pallas as pl
from jax.experimental.pallas import tpu as pltpu
from jax.experimental.pallas import tpu_sc as plsc

NC = 2
NS = 16
NW = NC * NS
L = 16
CHUNK = 4096


def _interp_kernel(nf, d, h, w, n, coords_hbm, vols_hbm, out_hbm,
                   coords_v, idx_refs, corner_refs, wz_v, wy_v, wx_v,
                   out_v, sem):
    b_per_w = n // NW
    n_chunks = b_per_w // CHUNK
    wid = lax.axis_index("s") * NC + lax.axis_index("c")
    base = wid * b_per_w

    hw = h * w
    dhw = d * hw

    def chunk_body(ci, _):
        cbase = base + ci * CHUNK
        pltpu.sync_copy(coords_hbm.at[pl.ds(cbase * 4, CHUNK * 4)], coords_v)

        lanes = lax.iota(jnp.int32, 16)

        def idx_body(i, _):
            q4 = (i * L + lanes) * 4
            zc = plsc.load_gather(coords_v, [q4])
            yc = plsc.load_gather(coords_v, [q4 + 1])
            xc = plsc.load_gather(coords_v, [q4 + 2])
            tc = plsc.load_gather(coords_v, [q4 + 3])
            sz = zc * float(d - 1)
            sy = yc * float(h - 1)
            sx = xc * float(w - 1)
            iz = sz.astype(jnp.int32)
            iy = sy.astype(jnp.int32)
            ix = sx.astype(jnp.int32)
            sl = pl.ds(i * L, L)
            wz_v[sl] = sz - iz.astype(jnp.float32)
            wy_v[sl] = sy - iy.astype(jnp.float32)
            wx_v[sl] = sx - ix.astype(jnp.float32)
            z0 = jnp.clip(iz, 0, d - 1)
            y0 = jnp.clip(iy, 0, h - 1)
            x0 = jnp.clip(ix, 0, w - 1)
            z1 = jnp.minimum(z0 + 1, d - 1)
            y1 = jnp.minimum(y0 + 1, h - 1)
            x1 = jnp.minimum(x0 + 1, w - 1)
            d0 = jnp.abs(tc + 1.0)
            d1 = jnp.abs(tc)
            d2 = jnp.abs(tc - 1.0)
            fi = jnp.where(d1 < d0,
                           jnp.where(d2 < d1, 2, 1),
                           jnp.where(d2 < d0, 2, 0)).astype(jnp.int32)
            b00 = fi * dhw + z0 * hw + y0 * w
            b01 = fi * dhw + z0 * hw + y1 * w
            b10 = fi * dhw + z1 * hw + y0 * w
            b11 = fi * dhw + z1 * hw + y1 * w
            idx_refs[0][sl] = b00 + x0
            idx_refs[1][sl] = b00 + x1
            idx_refs[2][sl] = b01 + x0
            idx_refs[3][sl] = b01 + x1
            idx_refs[4][sl] = b10 + x0
            idx_refs[5][sl] = b10 + x1
            idx_refs[6][sl] = b11 + x0
            idx_refs[7][sl] = b11 + x1
            return 0

        lax.fori_loop(0, CHUNK // L, idx_body, 0)

        copies = [pltpu.async_copy(vols_hbm.at[idx_refs[k]], corner_refs[k],
                                   sem) for k in range(8)]
        for c in copies:
            c.wait()

        def comb_body(i, _):
            sl = pl.ds(i * L, L)
            wx = wx_v[sl]
            wy = wy_v[sl]
            wz = wz_v[sl]
            c000 = corner_refs[0][sl]
            c001 = corner_refs[1][sl]
            c010 = corner_refs[2][sl]
            c011 = corner_refs[3][sl]
            c100 = corner_refs[4][sl]
            c101 = corner_refs[5][sl]
            c110 = corner_refs[6][sl]
            c111 = corner_refs[7][sl]
            c00 = c000 * (1 - wx) + c001 * wx
            c01 = c010 * (1 - wx) + c011 * wx
            c10 = c100 * (1 - wx) + c101 * wx
            c11 = c110 * (1 - wx) + c111 * wx
            c0 = c00 * (1 - wy) + c01 * wy
            c1 = c10 * (1 - wy) + c11 * wy
            out_v[sl] = c0 * (1 - wz) + c1 * wz
            return 0

        lax.fori_loop(0, CHUNK // L, comb_body, 0)
        pltpu.sync_copy(out_v, out_hbm.at[pl.ds(cbase, CHUNK)])
        return 0

    lax.fori_loop(0, n_chunks, chunk_body, 0)


def kernel(coords, vols):
    n = coords.shape[0]
    nf, d, h, w = vols.shape
    coords_flat = coords.reshape(-1)
    vols_flat = vols.reshape(-1)

    mesh = plsc.VectorSubcoreMesh(core_axis_name="c", subcore_axis_name="s",
                                  num_cores=NC, num_subcores=NS)
    body = functools.partial(_interp_kernel, nf, d, h, w, n)
    run = pl.kernel(
        body,
        out_type=jax.ShapeDtypeStruct((n,), jnp.float32),
        mesh=mesh,
        scratch_types=[
            pltpu.VMEM((4 * CHUNK,), jnp.float32),
            [pltpu.VMEM((CHUNK,), jnp.int32) for _ in range(8)],
            [pltpu.VMEM((CHUNK,), jnp.float32) for _ in range(8)],
            pltpu.VMEM((CHUNK,), jnp.float32),
            pltpu.VMEM((CHUNK,), jnp.float32),
            pltpu.VMEM((CHUNK,), jnp.float32),
            pltpu.VMEM((CHUNK,), jnp.float32),
            pltpu.SemaphoreType.DMA,
        ],
    )
    out = run(coords_flat, vols_flat)
    return out.reshape(n, 1)


# speedup vs baseline: 1.1409x; 1.1409x over previous
import functools

import jax
import jax.numpy as jnp
from jax import lax
from jax.experimental import pallas as pl
from jax.experimental.pallas import tpu as pltpu
from jax.experimental.pallas import tpu_sc as plsc

NC = 2
NS = 16
NW = NC * NS
L = 16
CHUNK = 4096


def _interp_kernel(nf, d, h, w, n, coords_hbm, vols_hbm, out_hbm,
                   zc_v, yc_v, xc_v, tc_v, idx_refs, corner_refs,
                   wz_v, wy_v, wx_v, out_v, sem):
    b_per_w = n // NW
    n_chunks = b_per_w // CHUNK
    wid = lax.axis_index("s") * NC + lax.axis_index("c")
    base = wid * b_per_w

    hw = h * w
    dhw = d * hw

    def chunk_body(ci, _):
        cbase = base + ci * CHUNK
        pltpu.sync_copy(coords_hbm.at[pl.ds(cbase, CHUNK)], zc_v)
        pltpu.sync_copy(coords_hbm.at[pl.ds(n + cbase, CHUNK)], yc_v)
        pltpu.sync_copy(coords_hbm.at[pl.ds(2 * n + cbase, CHUNK)], xc_v)
        pltpu.sync_copy(coords_hbm.at[pl.ds(3 * n + cbase, CHUNK)], tc_v)

        def idx_body(i, _):
            sl = pl.ds(i * L, L)
            zc = zc_v[sl]
            yc = yc_v[sl]
            xc = xc_v[sl]
            tc = tc_v[sl]
            sz = zc * float(d - 1)
            sy = yc * float(h - 1)
            sx = xc * float(w - 1)
            iz = sz.astype(jnp.int32)
            iy = sy.astype(jnp.int32)
            ix = sx.astype(jnp.int32)
            wz_v[sl] = sz - iz.astype(jnp.float32)
            wy_v[sl] = sy - iy.astype(jnp.float32)
            wx_v[sl] = sx - ix.astype(jnp.float32)
            z0 = jnp.clip(iz, 0, d - 1)
            y0 = jnp.clip(iy, 0, h - 1)
            x0 = jnp.clip(ix, 0, w - 1)
            z1 = jnp.minimum(z0 + 1, d - 1)
            y1 = jnp.minimum(y0 + 1, h - 1)
            x1 = jnp.minimum(x0 + 1, w - 1)
            d0 = jnp.abs(tc + 1.0)
            d1 = jnp.abs(tc)
            d2 = jnp.abs(tc - 1.0)
            fi = jnp.where(d1 < d0,
                           jnp.where(d2 < d1, 2, 1),
                           jnp.where(d2 < d0, 2, 0)).astype(jnp.int32)
            b00 = fi * dhw + z0 * hw + y0 * w
            b01 = fi * dhw + z0 * hw + y1 * w
            b10 = fi * dhw + z1 * hw + y0 * w
            b11 = fi * dhw + z1 * hw + y1 * w
            idx_refs[0][sl] = b00 + x0
            idx_refs[1][sl] = b00 + x1
            idx_refs[2][sl] = b01 + x0
            idx_refs[3][sl] = b01 + x1
            idx_refs[4][sl] = b10 + x0
            idx_refs[5][sl] = b10 + x1
            idx_refs[6][sl] = b11 + x0
            idx_refs[7][sl] = b11 + x1
            return 0

        lax.fori_loop(0, CHUNK // L, idx_body, 0)

        copies = [pltpu.async_copy(vols_hbm.at[idx_refs[k]], corner_refs[k],
                                   sem) for k in range(8)]
        for c in copies:
            c.wait()

        def comb_body(i, _):
            sl = pl.ds(i * L, L)
            wx = wx_v[sl]
            wy = wy_v[sl]
            wz = wz_v[sl]
            c000 = corner_refs[0][sl]
            c001 = corner_refs[1][sl]
            c010 = corner_refs[2][sl]
            c011 = corner_refs[3][sl]
            c100 = corner_refs[4][sl]
            c101 = corner_refs[5][sl]
            c110 = corner_refs[6][sl]
            c111 = corner_refs[7][sl]
            c00 = c000 * (1 - wx) + c001 * wx
            c01 = c010 * (1 - wx) + c011 * wx
            c10 = c100 * (1 - wx) + c101 * wx
            c11 = c110 * (1 - wx) + c111 * wx
            c0 = c00 * (1 - wy) + c01 * wy
            c1 = c10 * (1 - wy) + c11 * wy
            out_v[sl] = c0 * (1 - wz) + c1 * wz
            return 0

        lax.fori_loop(0, CHUNK // L, comb_body, 0)
        pltpu.sync_copy(out_v, out_hbm.at[pl.ds(cbase, CHUNK)])
        return 0

    lax.fori_loop(0, n_chunks, chunk_body, 0)


def kernel(coords, vols):
    n = coords.shape[0]
    nf, d, h, w = vols.shape
    coords_flat = coords.T.reshape(-1)
    vols_flat = vols.reshape(-1)

    mesh = plsc.VectorSubcoreMesh(core_axis_name="c", subcore_axis_name="s",
                                  num_cores=NC, num_subcores=NS)
    body = functools.partial(_interp_kernel, nf, d, h, w, n)
    run = pl.kernel(
        body,
        out_type=jax.ShapeDtypeStruct((n,), jnp.float32),
        mesh=mesh,
        scratch_types=[
            pltpu.VMEM((CHUNK,), jnp.float32),
            pltpu.VMEM((CHUNK,), jnp.float32),
            pltpu.VMEM((CHUNK,), jnp.float32),
            pltpu.VMEM((CHUNK,), jnp.float32),
            [pltpu.VMEM((CHUNK,), jnp.int32) for _ in range(8)],
            [pltpu.VMEM((CHUNK,), jnp.float32) for _ in range(8)],
            pltpu.VMEM((CHUNK,), jnp.float32),
            pltpu.VMEM((CHUNK,), jnp.float32),
            pltpu.VMEM((CHUNK,), jnp.float32),
            pltpu.VMEM((CHUNK,), jnp.float32),
            pltpu.SemaphoreType.DMA,
        ],
    )
    out = run(coords_flat, vols_flat)
    return out.reshape(n, 1)


# speedup vs baseline: 1.2529x; 1.0982x over previous
import functools

import jax
import jax.numpy as jnp
from jax import lax
from jax.experimental import pallas as pl
from jax.experimental.pallas import tpu as pltpu
from jax.experimental.pallas import tpu_sc as plsc

NC = 2
NS = 16
NW = NC * NS
L = 16
CHUNK = 2048


def _interp_kernel(nf, d, h, w, n, coords_hbm, vols_hbm, out_hbm,
                   zc_v, yc_v, xc_v, tc_v, idx_refs, corner_refs, w_refs,
                   out_v, gsem0, gsem1, csem):
    b_per_w = n // NW
    n_chunks = b_per_w // CHUNK
    n_pairs = n_chunks // 2
    wid = lax.axis_index("s") * NC + lax.axis_index("c")
    base = wid * b_per_w

    hw = h * w
    dhw = d * hw
    gsems = [gsem0, gsem1]

    def compute_and_fire(ci, p):
        cbase = base + ci * CHUNK
        cps = [
            pltpu.async_copy(coords_hbm.at[pl.ds(q * n + cbase, CHUNK)],
                             dst, csem)
            for q, dst in enumerate((zc_v, yc_v, xc_v, tc_v))
        ]
        for c in cps:
            c.wait()

        iref = idx_refs[p]
        wzr, wyr, wxr = w_refs[p]

        def idx_body(i, _):
            sl = pl.ds(i * L, L)
            zc = zc_v[sl]
            yc = yc_v[sl]
            xc = xc_v[sl]
            tc = tc_v[sl]
            sz = zc * float(d - 1)
            sy = yc * float(h - 1)
            sx = xc * float(w - 1)
            iz = sz.astype(jnp.int32)
            iy = sy.astype(jnp.int32)
            ix = sx.astype(jnp.int32)
            wzr[sl] = sz - iz.astype(jnp.float32)
            wyr[sl] = sy - iy.astype(jnp.float32)
            wxr[sl] = sx - ix.astype(jnp.float32)
            z0 = jnp.clip(iz, 0, d - 1)
            y0 = jnp.clip(iy, 0, h - 1)
            x0 = jnp.clip(ix, 0, w - 1)
            z1 = jnp.minimum(z0 + 1, d - 1)
            y1 = jnp.minimum(y0 + 1, h - 1)
            x1 = jnp.minimum(x0 + 1, w - 1)
            d0 = jnp.abs(tc + 1.0)
            d1 = jnp.abs(tc)
            d2 = jnp.abs(tc - 1.0)
            fi = jnp.where(d1 < d0,
                           jnp.where(d2 < d1, 2, 1),
                           jnp.where(d2 < d0, 2, 0)).astype(jnp.int32)
            b00 = fi * dhw + z0 * hw + y0 * w
            b01 = b00 + (y1 - y0) * w
            b10 = b00 + (z1 - z0) * hw
            b11 = b10 + (y1 - y0) * w
            iref[0][sl] = b00 + x0
            iref[1][sl] = b00 + x1
            iref[2][sl] = b01 + x0
            iref[3][sl] = b01 + x1
            iref[4][sl] = b10 + x0
            iref[5][sl] = b10 + x1
            iref[6][sl] = b11 + x0
            iref[7][sl] = b11 + x1
            return 0

        lax.fori_loop(0, CHUNK // L, idx_body, 0)
        for k in range(8):
            pltpu.async_copy(vols_hbm.at[iref[k]], corner_refs[p][k],
                             gsems[p])

    def drain_combine_store(ci, p):
        for k in range(8):
            pltpu.make_async_copy(vols_hbm.at[idx_refs[p][k]],
                                  corner_refs[p][k], gsems[p]).wait()
        cref = corner_refs[p]
        wzr, wyr, wxr = w_refs[p]

        def comb_body(i, _):
            sl = pl.ds(i * L, L)
            wx = wxr[sl]
            wy = wyr[sl]
            wz = wzr[sl]
            c00 = cref[0][sl] * (1 - wx) + cref[1][sl] * wx
            c01 = cref[2][sl] * (1 - wx) + cref[3][sl] * wx
            c10 = cref[4][sl] * (1 - wx) + cref[5][sl] * wx
            c11 = cref[6][sl] * (1 - wx) + cref[7][sl] * wx
            c0 = c00 * (1 - wy) + c01 * wy
            c1 = c10 * (1 - wy) + c11 * wy
            out_v[sl] = c0 * (1 - wz) + c1 * wz
            return 0

        lax.fori_loop(0, CHUNK // L, comb_body, 0)
        cbase = base + ci * CHUNK
        pltpu.sync_copy(out_v, out_hbm.at[pl.ds(cbase, CHUNK)])

    compute_and_fire(0, 0)

    def pair_body(pi, _):
        ci = pi * 2
        compute_and_fire(ci + 1, 1)
        drain_combine_store(ci, 0)
        compute_and_fire(ci + 2, 0)
        drain_combine_store(ci + 1, 1)
        return 0

    lax.fori_loop(0, n_pairs - 1, pair_body, 0)

    ci = (n_pairs - 1) * 2
    compute_and_fire(ci + 1, 1)
    drain_combine_store(ci, 0)
    drain_combine_store(ci + 1, 1)


def kernel(coords, vols):
    n = coords.shape[0]
    nf, d, h, w = vols.shape
    coords_flat = coords.T.reshape(-1)
    vols_flat = vols.reshape(-1)

    mesh = plsc.VectorSubcoreMesh(core_axis_name="c", subcore_axis_name="s",
                                  num_cores=NC, num_subcores=NS)
    body = functools.partial(_interp_kernel, nf, d, h, w, n)
    run = pl.kernel(
        body,
        out_type=jax.ShapeDtypeStruct((n,), jnp.float32),
        mesh=mesh,
        scratch_types=[
            pltpu.VMEM((CHUNK,), jnp.float32),
            pltpu.VMEM((CHUNK,), jnp.float32),
            pltpu.VMEM((CHUNK,), jnp.float32),
            pltpu.VMEM((CHUNK,), jnp.float32),
            [[pltpu.VMEM((CHUNK,), jnp.int32) for _ in range(8)]
             for _ in range(2)],
            [[pltpu.VMEM((CHUNK,), jnp.float32) for _ in range(8)]
             for _ in range(2)],
            [[pltpu.VMEM((CHUNK,), jnp.float32) for _ in range(3)]
             for _ in range(2)],
            pltpu.VMEM((CHUNK,), jnp.float32),
            pltpu.SemaphoreType.DMA,
            pltpu.SemaphoreType.DMA,
            pltpu.SemaphoreType.DMA,
        ],
    )
    out = run(coords_flat, vols_flat)
    return out.reshape(n, 1)
